# Initial kernel scaffold; baseline (speedup 1.0000x reference)
import jax
import jax.numpy as jnp
from jax.experimental import pallas as pl


def kernel(indices, tables):
    raise NotImplementedError("write your pallas kernel here")


# speedup vs baseline: 1.2905x; 1.2905x over previous
import functools

import jax
import jax.numpy as jnp
from jax import lax
from jax.experimental import pallas as pl
from jax.experimental.pallas import tpu as pltpu
from jax.experimental.pallas import tpu_sc as plsc

F = 26
B = 4096
L = 20
D = 128
V = 100000

_info = plsc.get_sparse_core_info()
NC, NS, LANES = _info.num_cores, _info.num_subcores, _info.num_lanes
NW = NC * NS
BAGS = B * F
BPW = BAGS // NW
NG = 16
NIT = BPW // NG
CH = 4
CHB = NG * L // CH
DCH = D // LANES


def _body(idx_hbm, tab_hbm, out_hbm, idx_v, rows_v, out_v, gsem):
    wid = lax.axis_index("s") * NC + lax.axis_index("c")
    idx_row0 = wid * (BPW * L // CHB)
    g0w = wid * BPW

    def it_body(it, carry):
        pltpu.sync_copy(idx_hbm.at[pl.ds(idx_row0 + it * CH, CH)], idx_v)
        for j in range(CH):
            pltpu.async_copy(
                tab_hbm.at[idx_v.at[j]],
                rows_v.at[pl.ds(j * CHB, CHB)],
                gsem,
            ).wait()

        def bag_body(jj, c2):
            base = jj * L
            for c in range(DCH):
                v = rows_v[base, pl.ds(c * LANES, LANES)]
                for l in range(1, L):
                    v = v + rows_v[base + l, pl.ds(c * LANES, LANES)]
                out_v[jj, pl.ds(c * LANES, LANES)] = v
            return c2

        lax.fori_loop(0, NG, bag_body, 0)
        pltpu.sync_copy(out_v, out_hbm.at[pl.ds(g0w + it * NG, NG)])
        return carry

    lax.fori_loop(0, NIT, it_body, 0)


_mesh = plsc.VectorSubcoreMesh(core_axis_name="c", subcore_axis_name="s")

_lookup = functools.partial(
    pl.kernel,
    mesh=_mesh,
    out_type=jax.ShapeDtypeStruct((BAGS, D), jnp.float32),
    scratch_types=[
        pltpu.VMEM((CH, CHB), jnp.int32),
        pltpu.VMEM((NG * L, D), jnp.float32),
        pltpu.VMEM((NG, D), jnp.float32),
        pltpu.SemaphoreType.DMA,
    ],
)(_body)


def kernel(indices, tables):
    idx = indices.astype(jnp.int32)
    idx = idx + (jnp.arange(F, dtype=jnp.int32) * V)[:, None, None]
    idx = jnp.transpose(idx, (1, 0, 2)).reshape(BAGS * L // CHB, CHB)
    tab = tables.reshape(F * V, D)
    out = _lookup(idx, tab)
    return out.reshape(B, F, D)


# speedup vs baseline: 2.4365x; 1.8881x over previous
import functools

import jax
import jax.numpy as jnp
from jax import lax
from jax.experimental import pallas as pl
from jax.experimental.pallas import tpu as pltpu
from jax.experimental.pallas import tpu_sc as plsc

F = 26
B = 4096
L = 20
D = 128
V = 100000

_info = plsc.get_sparse_core_info()
NC, NS, LANES = _info.num_cores, _info.num_subcores, _info.num_lanes
NW = NC * NS
BAGS = B * F
BPW = BAGS // NW
NG = 16
NIT = BPW // NG
CH = 4
CHB = NG * L // CH
DCH = D // LANES


def _body(idx_hbm, tab_hbm, out_hbm, idx_v, rows_v, out_v,
          gsem0, gsem1, osem0, osem1):
    wid = lax.axis_index("s") * NC + lax.axis_index("c")
    idx_row0 = wid * (BPW * L // CHB)
    g0w = wid * BPW
    gsems = (gsem0, gsem1)
    osems = (osem0, osem1)

    def load_idx(it, slot):
        pltpu.sync_copy(idx_hbm.at[pl.ds(idx_row0 + it * CH, CH)],
                        idx_v.at[slot])

    def gather_descr(slot, j):
        return pltpu.make_async_copy(
            tab_hbm.at[idx_v.at[slot, j]],
            rows_v.at[slot, pl.ds(j * CHB, CHB)],
            gsems[slot],
        )

    def out_descr(it, slot):
        return pltpu.make_async_copy(
            out_v.at[slot],
            out_hbm.at[pl.ds(g0w + it * NG, NG)],
            osems[slot],
        )

    def accumulate(slot):
        def bag_body(jj, c2):
            base = jj * L
            for c in range(DCH):
                v = rows_v[slot, base, pl.ds(c * LANES, LANES)]
                for l in range(1, L):
                    v = v + rows_v[slot, base + l, pl.ds(c * LANES, LANES)]
                out_v[slot, jj, pl.ds(c * LANES, LANES)] = v
            return c2

        lax.fori_loop(0, NG, bag_body, 0)

    load_idx(0, 0)
    for j in range(CH):
        gather_descr(0, j).start()

    def half(it, s):
        cur = it + s
        nslot = 1 - s

        @pl.when(cur + 1 < NIT)
        def _():
            load_idx(cur + 1, nslot)
            for j in range(CH):
                gather_descr(nslot, j).start()

        for j in range(CH):
            gather_descr(s, j).wait()

        @pl.when(cur >= 2)
        def _():
            out_descr(cur - 2, s).wait()

        accumulate(s)
        out_descr(cur, s).start()

    def it_body(i, carry):
        half(2 * i, 0)
        half(2 * i, 1)
        return carry

    lax.fori_loop(0, NIT // 2, it_body, 0)
    out_descr(NIT - 2, 0).wait()
    out_descr(NIT - 1, 1).wait()


_mesh = plsc.VectorSubcoreMesh(core_axis_name="c", subcore_axis_name="s")

_lookup = functools.partial(
    pl.kernel,
    mesh=_mesh,
    out_type=jax.ShapeDtypeStruct((BAGS, D), jnp.float32),
    scratch_types=[
        pltpu.VMEM((2, CH, CHB), jnp.int32),
        pltpu.VMEM((2, NG * L, D), jnp.float32),
        pltpu.VMEM((2, NG, D), jnp.float32),
        pltpu.SemaphoreType.DMA,
        pltpu.SemaphoreType.DMA,
        pltpu.SemaphoreType.DMA,
        pltpu.SemaphoreType.DMA,
    ],
)(_body)


def kernel(indices, tables):
    idx = indices.astype(jnp.int32)
    idx = idx + (jnp.arange(F, dtype=jnp.int32) * V)[:, None, None]
    idx = jnp.transpose(idx, (1, 0, 2)).reshape(BAGS * L // CHB, CHB)
    tab = tables.reshape(F * V, D)
    out = _lookup(idx, tab)
    return out.reshape(B, F, D)


# speedup vs baseline: 2.8133x; 1.1547x over previous
import functools

import jax
import jax.numpy as jnp
from jax import lax
from jax.experimental import pallas as pl
from jax.experimental.pallas import tpu as pltpu
from jax.experimental.pallas import tpu_sc as plsc

F = 26
B = 4096
L = 20
D = 128
V = 100000

_info = plsc.get_sparse_core_info()
NC, NS, LANES = _info.num_cores, _info.num_subcores, _info.num_lanes
NW = NC * NS
BAGS = B * F
BPW = BAGS // NW
NG = 16
NIT = BPW // NG
CH = 4
CHB = NG * L // CH
DCH = D // LANES
IDX_ROWS = BAGS * L // CHB


def _body(idx_hbm, tab_hbm, out_hbm, idx_v, rows_v, out_v, oidx_v,
          gsem0, gsem1, osem0, osem1, isem0, isem1):
    wid = lax.axis_index("s") * NC + lax.axis_index("c")
    idx_row0 = wid * (BPW * L // CHB)
    g0w = wid * BPW
    gsems = (gsem0, gsem1)
    osems = (osem0, osem1)
    isems = (isem0, isem1)

    def idx_descr(it, slot):
        return pltpu.make_async_copy(
            idx_hbm.at[pl.ds(idx_row0 + it * CH, CH)],
            idx_v.at[slot],
            isems[slot],
        )

    def gather_descr(slot, j):
        return pltpu.make_async_copy(
            tab_hbm.at[idx_v.at[slot, j]],
            rows_v.at[slot, pl.ds(j * CHB, CHB)],
            gsems[slot],
        )

    def out_descr(slot):
        return pltpu.make_async_copy(
            out_v.at[slot],
            out_hbm.at[oidx_v.at[slot]],
            osems[slot],
        )

    def globalize(it, slot):
        fv = ((g0w + it * NG) // B) * V
        fvv = jnp.full((LANES,), fv, dtype=jnp.int32)
        for j in range(CH):
            for k in range(CHB // LANES):
                sl = pl.ds(k * LANES, LANES)
                idx_v[slot, j, sl] = idx_v[slot, j, sl] + fvv

    def accumulate(it, slot):
        def bag_body(jj, c2):
            base = jj * L
            for c in range(DCH):
                v = rows_v[slot, base, pl.ds(c * LANES, LANES)]
                for l in range(1, L):
                    v = v + rows_v[slot, base + l, pl.ds(c * LANES, LANES)]
                out_v[slot, jj, pl.ds(c * LANES, LANES)] = v
            return c2

        lax.fori_loop(0, NG, bag_body, 0)
        g0 = g0w + it * NG
        obase = (g0 % B) * F + g0 // B
        oidx_v[slot, :] = obase + F * lax.iota(jnp.int32, LANES)

    idx_descr(0, 0).start()
    idx_descr(0, 0).wait()
    globalize(0, 0)
    for j in range(CH):
        gather_descr(0, j).start()
    idx_descr(1, 1).start()

    def half(it, s):
        cur = it + s
        ns = 1 - s

        for j in range(CH):
            gather_descr(s, j).wait()

        @pl.when(cur + 2 < NIT)
        def _():
            idx_descr(cur + 2, s).start()

        @pl.when(cur + 1 < NIT)
        def _():
            idx_descr(cur + 1, ns).wait()
            globalize(cur + 1, ns)
            for j in range(CH):
                gather_descr(ns, j).start()

        @pl.when(cur >= 2)
        def _():
            out_descr(s).wait()

        accumulate(cur, s)
        out_descr(s).start()

    def it_body(i, carry):
        half(2 * i, 0)
        half(2 * i, 1)
        return carry

    lax.fori_loop(0, NIT // 2, it_body, 0)
    out_descr(0).wait()
    out_descr(1).wait()


_mesh = plsc.VectorSubcoreMesh(core_axis_name="c", subcore_axis_name="s")

_lookup = functools.partial(
    pl.kernel,
    mesh=_mesh,
    out_type=jax.ShapeDtypeStruct((BAGS, D), jnp.float32),
    scratch_types=[
        pltpu.VMEM((2, CH, CHB), jnp.int32),
        pltpu.VMEM((2, NG * L, D), jnp.float32),
        pltpu.VMEM((2, NG, D), jnp.float32),
        pltpu.VMEM((2, LANES), jnp.int32),
        pltpu.SemaphoreType.DMA,
        pltpu.SemaphoreType.DMA,
        pltpu.SemaphoreType.DMA,
        pltpu.SemaphoreType.DMA,
        pltpu.SemaphoreType.DMA,
        pltpu.SemaphoreType.DMA,
    ],
)(_body)


def kernel(indices, tables):
    idx = indices.astype(jnp.int32).reshape(IDX_ROWS, CHB)
    tab = tables.reshape(F * V, D)
    out = _lookup(idx, tab)
    return out.reshape(B, F, D)
